# baseline (speedup 1.0000x reference)
import functools

import numpy as np
import jax
import jax.numpy as jnp
from jax import lax
from jax.experimental import pallas as pl
from jax.experimental.pallas import tpu as pltpu
from jax.experimental.pallas import tpu_sc as plsc

N = 4096
D = 128
E = 131072
G = 64
NH = 4
HD = D // NH
DT = 144
NC, NS, L = 2, 16, 16
NW = NC * NS
EPW = E // NW
CH = 128
NCHUNK = EPW // CH
RPS = N // NS
BQ = 512


NBUF = 4


def _build_segsum(interpret=False):
    mesh = plsc.VectorSubcoreMesh(
        core_axis_name="c", subcore_axis_name="s",
        num_cores=NC, num_subcores=NS)

    @functools.partial(
        pl.kernel,
        out_type=jax.ShapeDtypeStruct((NC, N, DT), jnp.float32),
        mesh=mesh,
        scratch_types=[
            pltpu.VMEM((NCHUNK, CH), jnp.int32),
            pltpu.VMEM((NCHUNK, CH), jnp.int32),
            pltpu.VMEM((NBUF, CH, DT), jnp.float32),
            pltpu.VMEM((16, DT), jnp.float32),
            pltpu.VMEM_SHARED((N, DT), jnp.float32),
        ] + [pltpu.SemaphoreType.DMA] * (2 * NBUF),
        compiler_params=pltpu.CompilerParams(use_tc_tiling_on_sc=False),
        interpret=interpret,
    )
    def segsum(table_hbm, src_hbm, dst_hbm, out_hbm,
               sidx_v, didx_v, rows_v, zrow_v, acc_sh, *sems):
        gsems, ssems = sems[:NBUF], sems[NBUF:]
        cid = lax.axis_index("c")
        sid = lax.axis_index("s")
        wid = sid * NC + cid

        for i in range(16):
            for j in range(DT // L):
                zrow_v[i, pl.ds(j * L, L)] = jnp.zeros((L,), jnp.float32)
        for r in range(RPS // 16):
            pltpu.sync_copy(zrow_v, acc_sh.at[pl.ds(sid * RPS + r * 16, 16), :])

        pltpu.sync_copy(src_hbm.at[pl.ds(wid * NCHUNK, NCHUNK), :], sidx_v)
        pltpu.sync_copy(dst_hbm.at[pl.ds(wid * NCHUNK, NCHUNK), :], didx_v)
        LA = NBUF // 2
        for b in range(LA):
            pltpu.async_copy(table_hbm.at[sidx_v.at[b]], rows_v.at[b], gsems[b])
        plsc.subcore_barrier()

        def gwait(b):
            pltpu.make_async_copy(table_hbm.at[pl.ds(0, CH), :],
                                  rows_v.at[b], gsems[b]).wait()

        def swait(b):
            pltpu.make_async_copy(rows_v.at[b], acc_sh.at[pl.ds(0, CH), :],
                                  ssems[b]).wait()

        def group(g, carry):
            for u in range(NBUF):
                i = g * NBUF + u
                gwait(u)
                pltpu.async_copy(rows_v.at[u], acc_sh.at[didx_v.at[i]],
                                 ssems[u], add=True)
                bn = (u + LA) % NBUF
                nxt = i + LA

                @pl.when(i >= LA)
                def _():
                    swait(bn)

                @pl.when(nxt < NCHUNK)
                def _():
                    pltpu.async_copy(table_hbm.at[sidx_v.at[nxt]],
                                     rows_v.at[bn], gsems[bn])
            return carry

        lax.fori_loop(0, NCHUNK // NBUF, group, 0)
        for j in range(LA):
            swait((NCHUNK - LA + j) % NBUF)
        plsc.subcore_barrier()
        pltpu.sync_copy(acc_sh.at[pl.ds(sid * RPS, RPS), :],
                        out_hbm.at[cid, pl.ds(sid * RPS, RPS), :])

    return segsum


def _dot_t(x, w):
    return lax.dot_general(x, w, (((1,), (1,)), ((), ())),
                           preferred_element_type=jnp.float32)


def _build_sage_linear(aug_out, interpret=False):
    dout = DT if aug_out else D

    def body(sums_ref, xaug_ref, wl_ref, bl_ref, wr_ref, o_ref):
        s = sums_ref[0, :, :] + sums_ref[1, :, :]
        cnt = jnp.maximum(s[:, D:D + 1], 1.0)
        agg = s[:, :D] / cnt
        x = xaug_ref[:, :D]
        h = jax.nn.relu(_dot_t(agg, wl_ref[...]) + bl_ref[...]
                        + _dot_t(x, wr_ref[...]))
        if aug_out:
            lane = lax.broadcasted_iota(jnp.int32, (BQ, DT - D), 1)
            o_ref[:, :D] = h
            o_ref[:, D:] = jnp.where(lane == 0, 1.0, 0.0)
        else:
            o_ref[...] = h

    return pl.pallas_call(
        body,
        grid=(N // BQ,),
        in_specs=[
            pl.BlockSpec((NC, BQ, DT), lambda i: (0, i, 0)),
            pl.BlockSpec((BQ, DT), lambda i: (i, 0)),
            pl.BlockSpec((D, D), lambda i: (0, 0)),
            pl.BlockSpec((1, D), lambda i: (0, 0)),
            pl.BlockSpec((D, D), lambda i: (0, 0)),
        ],
        out_specs=pl.BlockSpec((BQ, dout), lambda i: (i, 0)),
        out_shape=jax.ShapeDtypeStruct((N, dout), jnp.float32),
        interpret=interpret,
    )


def _build_proj(interpret=False):
    scale = 1.0 / np.sqrt(HD)

    def body(hm_ref, hp_ref,
             wq1, bq1, wk1, bk1, wv1, bv1,
             wq2, bq2, wk2, bk2, wv2, bv2,
             oqm, okp, ovp, oqp, okm, ovm):
        hm = hm_ref[...]
        hp = hp_ref[...]
        def vaug(x):
            one = jnp.ones((x.shape[0], 1), jnp.float32)
            parts = []
            for h in range(NH):
                parts += [x[:, h * HD:(h + 1) * HD], one]
            return jnp.concatenate(parts, axis=1)

        oqm[...] = ((_dot_t(hm, wq1[...]) + bq1[...]) * scale).astype(jnp.bfloat16)
        okp[...] = (_dot_t(hp, wk1[...]) + bk1[...]).astype(jnp.bfloat16)
        ovp[...] = vaug(_dot_t(hp, wv1[...]) + bv1[...]).astype(jnp.bfloat16)
        oqp[...] = ((_dot_t(hp, wq2[...]) + bq2[...]) * scale).astype(jnp.bfloat16)
        okm[...] = (_dot_t(hm, wk2[...]) + bk2[...]).astype(jnp.bfloat16)
        ovm[...] = vaug(_dot_t(hm, wv2[...]) + bv2[...]).astype(jnp.bfloat16)

    row = pl.BlockSpec((BQ, D), lambda i: (i, 0))
    rowv = pl.BlockSpec((BQ, NH * (HD + 1)), lambda i: (i, 0))
    wspec = pl.BlockSpec((D, D), lambda i: (0, 0))
    bspec = pl.BlockSpec((1, D), lambda i: (0, 0))
    qk_shape = jax.ShapeDtypeStruct((N, D), jnp.bfloat16)
    v_shape = jax.ShapeDtypeStruct((N, NH * (HD + 1)), jnp.bfloat16)
    return pl.pallas_call(
        body,
        grid=(N // BQ,),
        in_specs=[row, row] + [wspec, bspec] * 6,
        out_specs=[row, row, rowv, row, row, rowv],
        out_shape=[qk_shape, qk_shape, v_shape, qk_shape, qk_shape, v_shape],
        interpret=interpret,
    )


def _build_attn(interpret=False):
    def body(q_ref, k_ref, v_ref, hq_ref, o_ref):
        q = q_ref[...]
        k = k_ref[...]
        v = v_ref[...]
        outs = []
        for h in range(NH):
            sl = slice(h * HD, (h + 1) * HD)
            slv = slice(h * (HD + 1), (h + 1) * (HD + 1))
            s = _dot_t(q[:, sl], k[:, sl]).astype(jnp.bfloat16)
            m = jnp.max(s, axis=1, keepdims=True)
            p16 = jnp.exp((s - m).astype(jnp.float32)).astype(jnp.bfloat16)
            oc = lax.dot_general(p16, v[:, slv], (((1,), (0,)), ((), ())),
                                 preferred_element_type=jnp.float32)
            outs.append(oc[:, :HD] / oc[:, HD:HD + 1])
        o_ref[...] = hq_ref[...] + jnp.concatenate(outs, axis=1)

    row16 = pl.BlockSpec((BQ, D), lambda i: (i, 0))
    full = pl.BlockSpec((N, D), lambda i: (0, 0))
    fullv = pl.BlockSpec((N, NH * (HD + 1)), lambda i: (0, 0))
    rowf = pl.BlockSpec((BQ, D), lambda i: (i, 0))
    return pl.pallas_call(
        body,
        grid=(N // BQ,),
        in_specs=[row16, full, fullv, rowf],
        out_specs=rowf,
        out_shape=jax.ShapeDtypeStruct((N, D), jnp.float32),
        interpret=interpret,
    )


def _build_pool_mlp(interpret=False):
    def body(hm_ref, hp_ref, mb_ref, pb_ref,
             w1_ref, b1_ref, w2_ref, b2_ref, o_ref):
        gids = lax.broadcasted_iota(jnp.int32, (G, N), 0)

        def seg_mean(h, ids):
            mask = (gids == ids).astype(jnp.float32)
            s = lax.dot_general(mask, h, (((1,), (0,)), ((), ())),
                                preferred_element_type=jnp.float32)
            c = jnp.maximum(jnp.sum(mask, axis=1, keepdims=True), 1.0)
            return s / c

        zm = seg_mean(hm_ref[...], mb_ref[...])
        zp = seg_mean(hp_ref[...], pb_ref[...])
        z = jnp.concatenate([zm, zp], axis=1)
        h1 = jax.nn.relu(_dot_t(z, w1_ref[...]) + b1_ref[...])
        logits = jnp.sum(h1 * w2_ref[...], axis=1, keepdims=True) + b2_ref[0, 0]
        o_ref[...] = 1.0 / (1.0 + jnp.exp(-logits))

    return pl.pallas_call(
        body,
        in_specs=[
            pl.BlockSpec((N, D), lambda: (0, 0)),
            pl.BlockSpec((N, D), lambda: (0, 0)),
            pl.BlockSpec((1, N), lambda: (0, 0)),
            pl.BlockSpec((1, N), lambda: (0, 0)),
            pl.BlockSpec((D, 2 * D), lambda: (0, 0)),
            pl.BlockSpec((1, D), lambda: (0, 0)),
            pl.BlockSpec((1, D), lambda: (0, 0)),
            pl.BlockSpec((1, 1), lambda: (0, 0)),
        ],
        out_specs=pl.BlockSpec((G, 1), lambda: (0, 0)),
        out_shape=jax.ShapeDtypeStruct((G, 1), jnp.float32),
        interpret=interpret,
    )


_SEGSUM = None
_SAGE_AUG = _build_sage_linear(aug_out=True)
_SAGE_PLAIN = _build_sage_linear(aug_out=False)
_PROJ = _build_proj()
_ATTN = _build_attn()
_POOL = _build_pool_mlp()


def _augment(x):
    one = jnp.ones((x.shape[0], 1), jnp.float32)
    pad = jnp.zeros((x.shape[0], DT - D - 1), jnp.float32)
    return jnp.concatenate([x, one, pad], axis=1)


def kernel(x_mol, x_prot, params, edge_index_mol, edge_index_prot,
           mol_batch, prot_batch):
    global _SEGSUM
    if _SEGSUM is None:
        _SEGSUM = _build_segsum()
    p = params
    sm = edge_index_mol[0].astype(jnp.int32).reshape(E // CH, CH)
    dm = edge_index_mol[1].astype(jnp.int32).reshape(E // CH, CH)
    sp = edge_index_prot[0].astype(jnp.int32).reshape(E // CH, CH)
    dp = edge_index_prot[1].astype(jnp.int32).reshape(E // CH, CH)

    xm = _augment(x_mol.astype(jnp.float32))
    xp = _augment(x_prot.astype(jnp.float32))

    def sage(x_aug, sums, pre, layer, aug):
        f = _SAGE_AUG if aug else _SAGE_PLAIN
        return f(sums, x_aug,
                 p[pre + layer + "_Wl"], p[pre + layer + "_bl"].reshape(1, D),
                 p[pre + layer + "_Wr"])

    sums_m1 = _SEGSUM(xm, sm, dm)
    sums_p1 = _SEGSUM(xp, sp, dp)
    hm1 = sage(xm, sums_m1, "mol", "1", True)
    hp1 = sage(xp, sums_p1, "prot", "1", True)
    sums_m2 = _SEGSUM(hm1, sm, dm)
    sums_p2 = _SEGSUM(hp1, sp, dp)
    hm = sage(hm1, sums_m2, "mol", "2", False)
    hp = sage(hp1, sums_p2, "prot", "2", False)

    qm, kp, vp, qp, km, vm = _PROJ(
        hm, hp,
        p["mp_WQ"], p["mp_bQ"].reshape(1, D),
        p["mp_WK"], p["mp_bK"].reshape(1, D),
        p["mp_WV"], p["mp_bV"].reshape(1, D),
        p["pm_WQ"], p["pm_bQ"].reshape(1, D),
        p["pm_WK"], p["pm_bK"].reshape(1, D),
        p["pm_WV"], p["pm_bV"].reshape(1, D),
    )
    hm2 = _ATTN(qm, kp, vp, hm)
    hp2 = _ATTN(qp, km, vm, hp)

    out = _POOL(hm2, hp2,
                mol_batch.reshape(1, N).astype(jnp.int32),
                prot_batch.reshape(1, N).astype(jnp.int32),
                p["fc1_W"], p["fc1_b"].reshape(1, D),
                p["fc2_W"], p["fc2_b"].reshape(1, 1))
    return out.reshape(G)


# speedup vs baseline: 1.0624x; 1.0624x over previous
import functools

import numpy as np
import jax
import jax.numpy as jnp
from jax import lax
from jax.experimental import pallas as pl
from jax.experimental.pallas import tpu as pltpu
from jax.experimental.pallas import tpu_sc as plsc

N = 4096
NT = 2 * N
D = 128
E = 131072
ET = 2 * E
G = 64
NH = 4
HD = D // NH
DV = NH * (HD + 1)
DT = 144
NC, NS, L = 2, 16, 16
NW = NC * NS
CH = 128
NCHUNK = E // (NS * CH)
RPS = N // NS
BQ = 512
GPB = N // BQ
NBUF = 4


def _build_segsum(interpret=False):
    mesh = plsc.VectorSubcoreMesh(
        core_axis_name="c", subcore_axis_name="s",
        num_cores=NC, num_subcores=NS)

    @functools.partial(
        pl.kernel,
        out_type=jax.ShapeDtypeStruct((NC, N, DT), jnp.float32),
        mesh=mesh,
        scratch_types=[
            pltpu.VMEM((NCHUNK, CH), jnp.int32),
            pltpu.VMEM((NCHUNK, CH), jnp.int32),
            pltpu.VMEM((NBUF, CH, DT), jnp.float32),
            pltpu.VMEM((16, DT), jnp.float32),
            pltpu.VMEM_SHARED((N, DT), jnp.float32),
        ] + [pltpu.SemaphoreType.DMA] * (2 * NBUF),
        compiler_params=pltpu.CompilerParams(use_tc_tiling_on_sc=False),
        interpret=interpret,
    )
    def segsum(table_hbm, src_hbm, dst_hbm, out_hbm,
               sidx_v, didx_v, rows_v, zrow_v, acc_sh, *sems):
        gsems, ssems = sems[:NBUF], sems[NBUF:]
        cid = lax.axis_index("c")
        sid = lax.axis_index("s")
        base = cid * (E // CH) + sid * NCHUNK

        for i in range(16):
            for j in range(DT // L):
                zrow_v[i, pl.ds(j * L, L)] = jnp.zeros((L,), jnp.float32)
        for r in range(RPS // 16):
            pltpu.sync_copy(zrow_v, acc_sh.at[pl.ds(sid * RPS + r * 16, 16), :])

        pltpu.sync_copy(src_hbm.at[pl.ds(base, NCHUNK), :], sidx_v)
        pltpu.sync_copy(dst_hbm.at[pl.ds(base, NCHUNK), :], didx_v)
        for b in range(NBUF):
            pltpu.async_copy(table_hbm.at[sidx_v.at[b]], rows_v.at[b], gsems[b])
        plsc.subcore_barrier()

        def group(g, carry):
            for b in range(NBUF):
                i = g * NBUF + b
                pltpu.make_async_copy(table_hbm.at[pl.ds(0, CH), :],
                                      rows_v.at[b], gsems[b]).wait()
                pltpu.async_copy(rows_v.at[b], acc_sh.at[didx_v.at[i]],
                                 ssems[b], add=True).wait()
                nxt = i + NBUF

                @pl.when(nxt < NCHUNK)
                def _():
                    pltpu.async_copy(table_hbm.at[sidx_v.at[nxt]],
                                     rows_v.at[b], gsems[b])
            return carry

        lax.fori_loop(0, NCHUNK // NBUF, group, 0)
        plsc.subcore_barrier()
        pltpu.sync_copy(acc_sh.at[pl.ds(sid * RPS, RPS), :],
                        out_hbm.at[cid, pl.ds(sid * RPS, RPS), :])

    return segsum


def _dot_t(x, w):
    return lax.dot_general(x, w, (((1,), (1,)), ((), ())),
                           preferred_element_type=jnp.float32)


def _build_sage_linear(aug_out, interpret=False):
    dout = DT if aug_out else D

    def body(sums_ref, xaug_ref, wl_ref, bl_ref, wr_ref, o_ref):
        s = sums_ref[...]
        cnt = jnp.maximum(s[:, D:D + 1], 1.0)
        agg = s[:, :D] / cnt
        x = xaug_ref[:, :D]
        h = jax.nn.relu(_dot_t(agg, wl_ref[0]) + bl_ref[0]
                        + _dot_t(x, wr_ref[0]))
        if aug_out:
            lane = lax.broadcasted_iota(jnp.int32, (BQ, DT - D), 1)
            o_ref[:, :D] = h
            o_ref[:, D:] = jnp.where(lane == 0, 1.0, 0.0)
        else:
            o_ref[...] = h

    return pl.pallas_call(
        body,
        grid=(NT // BQ,),
        in_specs=[
            pl.BlockSpec((BQ, DT), lambda i: (i, 0)),
            pl.BlockSpec((BQ, DT), lambda i: (i, 0)),
            pl.BlockSpec((1, D, D), lambda i: (i // GPB, 0, 0)),
            pl.BlockSpec((1, 1, D), lambda i: (i // GPB, 0, 0)),
            pl.BlockSpec((1, D, D), lambda i: (i // GPB, 0, 0)),
        ],
        out_specs=pl.BlockSpec((BQ, dout), lambda i: (i, 0)),
        out_shape=jax.ShapeDtypeStruct((NT, dout), jnp.float32),
        interpret=interpret,
    )


def _build_attn(interpret=False):
    scale = 1.0 / np.sqrt(HD)

    def body(a_ref, hkv_ref, wq, bq, wk, bk, wv, bv, o_ref, k_sc, v_sc):
        blk = pl.program_id(1)

        @pl.when(blk == 0)
        def _():
            kn = hkv_ref[0]
            k_sc[...] = (_dot_t(kn, wk[0]) + bk[0]).astype(jnp.bfloat16)
            vfull = _dot_t(kn, wv[0]) + bv[0]
            one = jnp.ones((N, 1), jnp.float32)
            parts = []
            for h in range(NH):
                parts += [vfull[:, h * HD:(h + 1) * HD], one]
            v_sc[...] = jnp.concatenate(parts, axis=1).astype(jnp.bfloat16)

        a = a_ref[...]
        q = ((_dot_t(a, wq[0]) + bq[0]) * scale).astype(jnp.bfloat16)
        k = k_sc[...]
        v = v_sc[...]
        outs = []
        for h in range(NH):
            sl = slice(h * HD, (h + 1) * HD)
            slv = slice(h * (HD + 1), (h + 1) * (HD + 1))
            s = _dot_t(q[:, sl], k[:, sl]).astype(jnp.bfloat16)
            m = jnp.max(s, axis=1, keepdims=True)
            p16 = jnp.exp((s - m).astype(jnp.float32)).astype(jnp.bfloat16)
            oc = lax.dot_general(p16, v[:, slv], (((1,), (0,)), ((), ())),
                                 preferred_element_type=jnp.float32)
            outs.append(oc[:, :HD] / oc[:, HD:HD + 1])
        o_ref[...] = a + jnp.concatenate(outs, axis=1)

    return pl.pallas_call(
        body,
        grid=(2, GPB),
        in_specs=[
            pl.BlockSpec((BQ, D), lambda d, i: (d * GPB + i, 0)),
            pl.BlockSpec((1, N, D), lambda d, i: (1 - d, 0, 0)),
            pl.BlockSpec((1, D, D), lambda d, i: (d, 0, 0)),
            pl.BlockSpec((1, 1, D), lambda d, i: (d, 0, 0)),
            pl.BlockSpec((1, D, D), lambda d, i: (d, 0, 0)),
            pl.BlockSpec((1, 1, D), lambda d, i: (d, 0, 0)),
            pl.BlockSpec((1, D, D), lambda d, i: (d, 0, 0)),
            pl.BlockSpec((1, 1, D), lambda d, i: (d, 0, 0)),
        ],
        out_specs=pl.BlockSpec((BQ, D), lambda d, i: (d * GPB + i, 0)),
        out_shape=jax.ShapeDtypeStruct((NT, D), jnp.float32),
        scratch_shapes=[
            pltpu.VMEM((N, D), jnp.bfloat16),
            pltpu.VMEM((N, DV), jnp.bfloat16),
        ],
        interpret=interpret,
    )


def _build_pool_mlp(interpret=False):
    def body(hm_ref, hp_ref, mb_ref, pb_ref,
             w1_ref, b1_ref, w2_ref, b2_ref, o_ref):
        gids = lax.broadcasted_iota(jnp.int32, (G, N), 0)

        def seg_mean(h, ids):
            mask = (gids == ids).astype(jnp.float32)
            s = lax.dot_general(mask, h, (((1,), (0,)), ((), ())),
                                preferred_element_type=jnp.float32)
            c = jnp.maximum(jnp.sum(mask, axis=1, keepdims=True), 1.0)
            return s / c

        zm = seg_mean(hm_ref[0], mb_ref[...])
        zp = seg_mean(hp_ref[0], pb_ref[...])
        z = jnp.concatenate([zm, zp], axis=1)
        h1 = jax.nn.relu(_dot_t(z, w1_ref[...]) + b1_ref[...])
        logits = jnp.sum(h1 * w2_ref[...], axis=1, keepdims=True) + b2_ref[0, 0]
        o_ref[...] = 1.0 / (1.0 + jnp.exp(-logits))

    return pl.pallas_call(
        body,
        grid=(1,),
        in_specs=[
            pl.BlockSpec((1, N, D), lambda i: (0, 0, 0)),
            pl.BlockSpec((1, N, D), lambda i: (1, 0, 0)),
            pl.BlockSpec((1, N), lambda i: (0, 0)),
            pl.BlockSpec((1, N), lambda i: (0, 0)),
            pl.BlockSpec((D, 2 * D), lambda i: (0, 0)),
            pl.BlockSpec((1, D), lambda i: (0, 0)),
            pl.BlockSpec((1, D), lambda i: (0, 0)),
            pl.BlockSpec((1, 1), lambda i: (0, 0)),
        ],
        out_specs=pl.BlockSpec((G, 1), lambda i: (0, 0)),
        out_shape=jax.ShapeDtypeStruct((G, 1), jnp.float32),
        interpret=interpret,
    )


_SEGSUM = None
_SAGE_AUG = _build_sage_linear(aug_out=True)
_SAGE_PLAIN = _build_sage_linear(aug_out=False)
_ATTN = _build_attn()
_POOL = _build_pool_mlp()


def _augment(x):
    one = jnp.ones((x.shape[0], 1), jnp.float32)
    pad = jnp.zeros((x.shape[0], DT - D - 1), jnp.float32)
    return jnp.concatenate([x, one, pad], axis=1)


def kernel(x_mol, x_prot, params, edge_index_mol, edge_index_prot,
           mol_batch, prot_batch):
    global _SEGSUM
    if _SEGSUM is None:
        _SEGSUM = _build_segsum()
    p = params

    src = jnp.concatenate([edge_index_mol[0].astype(jnp.int32),
                           edge_index_prot[0].astype(jnp.int32) + N]
                          ).reshape(ET // CH, CH)
    dst = jnp.concatenate([edge_index_mol[1].astype(jnp.int32),
                           edge_index_prot[1].astype(jnp.int32)]
                          ).reshape(ET // CH, CH)
    x_all = jnp.concatenate([_augment(x_mol.astype(jnp.float32)),
                             _augment(x_prot.astype(jnp.float32))], axis=0)

    def stk(*names):
        return jnp.stack([p[n] for n in names])

    h1 = _SAGE_AUG(_SEGSUM(x_all, src, dst).reshape(NT, DT), x_all,
                   stk("mol1_Wl", "prot1_Wl"),
                   stk("mol1_bl", "prot1_bl").reshape(2, 1, D),
                   stk("mol1_Wr", "prot1_Wr"))
    h2 = _SAGE_PLAIN(_SEGSUM(h1, src, dst).reshape(NT, DT), h1,
                     stk("mol2_Wl", "prot2_Wl"),
                     stk("mol2_bl", "prot2_bl").reshape(2, 1, D),
                     stk("mol2_Wr", "prot2_Wr"))

    h2attn = _ATTN(h2, h2.reshape(2, N, D),
                   stk("mp_WQ", "pm_WQ"), stk("mp_bQ", "pm_bQ").reshape(2, 1, D),
                   stk("mp_WK", "pm_WK"), stk("mp_bK", "pm_bK").reshape(2, 1, D),
                   stk("mp_WV", "pm_WV"), stk("mp_bV", "pm_bV").reshape(2, 1, D))

    hst = h2attn.reshape(2, N, D)
    out = _POOL(hst, hst,
                mol_batch.reshape(1, N).astype(jnp.int32),
                prot_batch.reshape(1, N).astype(jnp.int32),
                p["fc1_W"], p["fc1_b"].reshape(1, D),
                p["fc2_W"], p["fc2_b"].reshape(1, 1))
    return out.reshape(G)
